# Initial kernel scaffold; baseline (speedup 1.0000x reference)
import functools

import jax
import jax.numpy as jnp
from jax.experimental import pallas as pl
from jax.experimental.pallas import tpu as pltpu

T = 2048
H = 768
E = 8
TOPK = 2
DFF = 384
NG = 4
TG = 2
RSF = 2.5


def _router_weights(x, gate_w, bias):
    logits = jax.lax.dot_general(
        x, gate_w, (((1,), (1,)), ((), ())), preferred_element_type=jnp.float32
    )
    scores = jax.nn.sigmoid(logits)
    s = scores + bias
    gsum = s[:, 0::2] + s[:, 1::2]
    grank = jnp.zeros_like(gsum)
    giota = jax.lax.broadcasted_iota(jnp.int32, gsum.shape, 1)
    for j in range(NG):
        cj = gsum[:, j : j + 1]
        grank += ((cj > gsum) | ((cj == gsum) & (j < giota))).astype(jnp.float32)
    gmask = (grank < TG).astype(jnp.float32)
    eiota = jax.lax.broadcasted_iota(jnp.int32, s.shape, 1)
    mask8 = jnp.zeros_like(s)
    for g in range(NG):
        mask8 += gmask[:, g : g + 1] * (eiota // (E // NG) == g).astype(jnp.float32)
    tmp = jnp.where(mask8 > 0, s, 0.0)
    erank = jnp.zeros_like(tmp)
    for j in range(E):
        cj = tmp[:, j : j + 1]
        erank += ((cj > tmp) | ((cj == tmp) & (j < eiota))).astype(jnp.float32)
    msel = (erank < TOPK).astype(jnp.float32)
    wun = msel * scores
    denom = jnp.sum(wun, axis=1, keepdims=True)
    return wun / denom * RSF


def _dense_body(x_ref, gw_ref, bias_ref, wgu_ref, wd_ref, out_ref, w_scr, xb_scr):
    e = pl.program_id(0)

    @pl.when(e == 0)
    def _():
        x = x_ref[...]
        w_scr[...] = _router_weights(x, gw_ref[...], bias_ref[...])
        xb_scr[...] = x.astype(jnp.bfloat16)

    xb = xb_scr[...]
    gu = jax.lax.dot_general(
        xb, wgu_ref[0], (((1,), (1,)), ((), ())), preferred_element_type=jnp.float32
    )
    g = gu[:, :DFF]
    u = gu[:, DFF:]
    h = (g * jax.nn.sigmoid(g) * u).astype(jnp.bfloat16)
    d = jax.lax.dot_general(
        h, wd_ref[0], (((1,), (1,)), ((), ())), preferred_element_type=jnp.float32
    )
    lane = jax.lax.broadcasted_iota(jnp.int32, (T, E), 1)
    wcol = jnp.sum(w_scr[...] * (lane == e).astype(jnp.float32), axis=1, keepdims=True)
    wcol = wcol + jnp.where(e == E, 1.0, 0.0)

    @pl.when(e == 0)
    def _():
        out_ref[...] = wcol * d

    @pl.when(e > 0)
    def _():
        out_ref[...] += wcol * d


def kernel(hidden_states, gate_W, e_score_correction_bias, We_gate_up, We_down,
           Ws_gate_up, Ws_down):
    wgu_all = jnp.concatenate(
        [We_gate_up, Ws_gate_up[None]], axis=0).astype(jnp.bfloat16)
    wd_all = jnp.concatenate(
        [We_down, Ws_down[None]], axis=0).astype(jnp.bfloat16)
    bias2d = e_score_correction_bias.reshape(1, E)

    return pl.pallas_call(
        _dense_body,
        grid=(E + 1,),
        in_specs=[
            pl.BlockSpec((T, H), lambda e: (0, 0)),
            pl.BlockSpec((E, H), lambda e: (0, 0)),
            pl.BlockSpec((1, E), lambda e: (0, 0)),
            pl.BlockSpec((1, 2 * DFF, H), lambda e: (e, 0, 0)),
            pl.BlockSpec((1, H, DFF), lambda e: (e, 0, 0)),
        ],
        out_specs=pl.BlockSpec((T, H), lambda e: (0, 0)),
        out_shape=jax.ShapeDtypeStruct((T, H), jnp.float32),
        scratch_shapes=[
            pltpu.VMEM((T, E), jnp.float32),
            pltpu.VMEM((T, H), jnp.bfloat16),
        ],
        compiler_params=pltpu.CompilerParams(
            dimension_semantics=("arbitrary",),
        ),
    )(hidden_states, gate_W, bias2d, wgu_all, wd_all)


# speedup vs baseline: 1.5358x; 1.5358x over previous
import functools

import jax
import jax.numpy as jnp
from jax.experimental import pallas as pl
from jax.experimental.pallas import tpu as pltpu

T = 2048
H = 768
E = 8
TOPK = 2
DFF = 384
NG = 4
TG = 2
RSF = 2.5


def _router_weights(x, gate_w, bias):
    logits = jax.lax.dot_general(
        x, gate_w, (((1,), (1,)), ((), ())), preferred_element_type=jnp.float32
    )
    scores = jax.nn.sigmoid(logits)
    s = scores + bias
    gsum = jnp.concatenate(
        [s[:, 2 * g : 2 * g + 1] + s[:, 2 * g + 1 : 2 * g + 2] for g in range(NG)],
        axis=1,
    )
    grank = jnp.zeros_like(gsum)
    giota = jax.lax.broadcasted_iota(jnp.int32, gsum.shape, 1)
    for j in range(NG):
        cj = gsum[:, j : j + 1]
        grank += ((cj > gsum) | ((cj == gsum) & (j < giota))).astype(jnp.float32)
    gmask = (grank < TG).astype(jnp.float32)
    eiota = jax.lax.broadcasted_iota(jnp.int32, s.shape, 1)
    mask8 = jnp.concatenate(
        [gmask[:, g : g + 1] for g in range(NG) for _ in range(E // NG)], axis=1
    )
    tmp = jnp.where(mask8 > 0, s, 0.0)
    erank = jnp.zeros_like(tmp)
    for j in range(E):
        cj = tmp[:, j : j + 1]
        erank += ((cj > tmp) | ((cj == tmp) & (j < eiota))).astype(jnp.float32)
    msel = (erank < TOPK).astype(jnp.float32)
    wun = msel * scores
    denom = jnp.sum(wun, axis=1, keepdims=True)
    return wun / denom * RSF


def _dense_body(x_ref, gw_ref, bias_ref, wgu_ref, wd_ref, out_ref, w_scr, xb_scr):
    e = pl.program_id(0)

    @pl.when(e == 0)
    def _():
        x = x_ref[...]
        w_scr[...] = _router_weights(x, gw_ref[...], bias_ref[...])
        xb_scr[...] = x.astype(jnp.bfloat16)

    xb = xb_scr[...]
    gu = jax.lax.dot_general(
        xb, wgu_ref[0], (((1,), (1,)), ((), ())), preferred_element_type=jnp.float32
    )
    g = gu[:, :DFF]
    u = gu[:, DFF:]
    h = (g * jax.nn.sigmoid(g) * u).astype(jnp.bfloat16)
    d = jax.lax.dot_general(
        h, wd_ref[0], (((1,), (1,)), ((), ())), preferred_element_type=jnp.float32
    )
    lane = jax.lax.broadcasted_iota(jnp.int32, (T, E), 1)
    wcol = jnp.sum(w_scr[...] * (lane == e).astype(jnp.float32), axis=1, keepdims=True)
    wcol = wcol + jnp.where(e == E, 1.0, 0.0)

    @pl.when(e == 0)
    def _():
        out_ref[...] = wcol * d

    @pl.when(e > 0)
    def _():
        out_ref[...] += wcol * d


def kernel(hidden_states, gate_W, e_score_correction_bias, We_gate_up, We_down,
           Ws_gate_up, Ws_down):
    wgu_all = jnp.concatenate(
        [We_gate_up, Ws_gate_up[None]], axis=0).astype(jnp.bfloat16)
    wd_all = jnp.concatenate(
        [We_down, Ws_down[None]], axis=0).astype(jnp.bfloat16)
    bias2d = e_score_correction_bias.reshape(1, E)

    return pl.pallas_call(
        _dense_body,
        grid=(E + 1,),
        in_specs=[
            pl.BlockSpec((T, H), lambda e: (0, 0)),
            pl.BlockSpec((E, H), lambda e: (0, 0)),
            pl.BlockSpec((1, E), lambda e: (0, 0)),
            pl.BlockSpec((1, 2 * DFF, H), lambda e: (e, 0, 0)),
            pl.BlockSpec((1, H, DFF), lambda e: (e, 0, 0)),
        ],
        out_specs=pl.BlockSpec((T, H), lambda e: (0, 0)),
        out_shape=jax.ShapeDtypeStruct((T, H), jnp.float32),
        scratch_shapes=[
            pltpu.VMEM((T, E), jnp.float32),
            pltpu.VMEM((T, H), jnp.bfloat16),
        ],
        compiler_params=pltpu.CompilerParams(
            dimension_semantics=("arbitrary",),
        ),
    )(hidden_states, gate_W, bias2d, wgu_all, wd_all)
